# baseline (speedup 1.0000x reference)
import functools

import jax
import jax.numpy as jnp
from jax import lax
from jax.experimental import pallas as pl
from jax.experimental.pallas import tpu as pltpu
from jax.experimental.pallas import tpu_sc as plsc

B = 1048576
NC = 2
NS = 16
NW = NC * NS
ROWS_PER_W = B // NW
CHUNK_ROWS = 2048
NCHUNK = ROWS_PER_W // CHUNK_ROWS
GROUPS = CHUNK_ROWS // 16


def _body(x_hbm, wb_hbm, vals_hbm, idx_hbm, wbuf, xbuf, vbuf, ibuf):
    cid = lax.axis_index("c")
    sid = lax.axis_index("s")
    wid = sid * NC + cid
    row0 = wid * ROWS_PER_W

    pltpu.sync_copy(wb_hbm, wbuf)

    iota = lax.iota(jnp.int32, 16)
    i20 = iota * 20
    i3 = iota * 3

    def bc(i):
        return wbuf[pl.ds(i * 16, 16)]

    def rne_bf16(v):
        u = plsc.bitcast(v, jnp.uint32)
        t = (u >> 16) & jnp.uint32(1)
        r = (u + jnp.uint32(0x7FFF) + t) & jnp.uint32(0xFFFF0000)
        return plsc.bitcast(r, jnp.float32)

    Wv = [[bc(j * 4 + k) for k in range(4)] for j in range(5)]
    Wbv = [[rne_bf16(Wv[m][k]) for k in range(4)] for m in range(5)]
    bv = [bc(20 + m) for m in range(5)]
    jconst = [jnp.full((16,), j, jnp.int32) for j in range(5)]
    zero = jnp.zeros((16,), jnp.float32)

    def do_group(base20, base3):
        s = []
        for j in range(5):
            xs = [
                rne_bf16(
                    plsc.load_gather(xbuf, [base20 + (4 * j + k)]) + Wv[j][k]
                )
                for k in range(4)
            ]
            sj = None
            for m in range(5):
                acc = xs[0] * Wbv[m][0]
                for k in range(1, 4):
                    acc = xs[k] * Wbv[m][k] + acc
                r = jnp.maximum(acc + bv[m], zero)
                sj = r if sj is None else sj + r
            s.append(sj)

        v = list(s)
        ix = list(jconst)

        def cex(p, q):
            c = v[q] > v[p]
            vp = jnp.where(c, v[q], v[p])
            vq = jnp.where(c, v[p], v[q])
            ip = jnp.where(c, ix[q], ix[p])
            iq = jnp.where(c, ix[p], ix[q])
            v[p], v[q] = vp, vq
            ix[p], ix[q] = ip, iq

        for p in (3, 2, 1, 0):
            cex(p, p + 1)
        for p in (3, 2, 1):
            cex(p, p + 1)
        for p in (3, 2):
            cex(p, p + 1)

        for t in range(3):
            plsc.store_scatter(vbuf, [base3 + t], v[t])
            plsc.store_scatter(ibuf, [base3 + t], ix[t])

    def do_chunk(c, carry):
        xoff = (row0 + c * CHUNK_ROWS) * 20
        pltpu.sync_copy(x_hbm.at[pl.ds(xoff, CHUNK_ROWS * 20)], xbuf)

        def grp(g, inner):
            do_group(i20 + g * 320, i3 + g * 48)
            return inner

        lax.fori_loop(0, GROUPS, grp, 0)

        ooff = (row0 + c * CHUNK_ROWS) * 3
        pltpu.sync_copy(vbuf, vals_hbm.at[pl.ds(ooff, CHUNK_ROWS * 3)])
        pltpu.sync_copy(ibuf, idx_hbm.at[pl.ds(ooff, CHUNK_ROWS * 3)])
        return carry

    lax.fori_loop(0, NCHUNK, do_chunk, 0)


@jax.jit
def _run(xflat, wb):
    mesh = plsc.VectorSubcoreMesh(
        core_axis_name="c", subcore_axis_name="s",
        num_cores=NC, num_subcores=NS,
    )
    f = pl.kernel(
        _body,
        out_type=[
            jax.ShapeDtypeStruct((B * 3,), jnp.float32),
            jax.ShapeDtypeStruct((B * 3,), jnp.int32),
        ],
        mesh=mesh,
        compiler_params=pltpu.CompilerParams(needs_layout_passes=False),
        scratch_types=[
            pltpu.VMEM((400,), jnp.float32),
            pltpu.VMEM((CHUNK_ROWS * 20,), jnp.float32),
            pltpu.VMEM((CHUNK_ROWS * 3,), jnp.float32),
            pltpu.VMEM((CHUNK_ROWS * 3,), jnp.int32),
        ],
    )
    return f(xflat, wb)


def kernel(x, W, b):
    xflat = x.reshape(B * 20)
    wb = jnp.repeat(jnp.concatenate([W.reshape(-1), b]), 16)
    vals, idxs = _run(xflat, wb)
    return vals.reshape(B, 3), idxs.reshape(B, 3)


# speedup vs baseline: 27.2391x; 27.2391x over previous
import jax
import jax.numpy as jnp
from jax import lax
from jax.experimental import pallas as pl
from jax.experimental.pallas import tpu as pltpu
from jax.experimental.pallas import tpu_sc as plsc

B = 1048576
NC = 2
NS = 16
NW = NC * NS
ROWS_PER_W = B // NW
CHUNK_ROWS = 2048
NCHUNK = ROWS_PER_W // CHUNK_ROWS
GROUPS = CHUNK_ROWS // 16
TILES = CHUNK_ROWS // 128
XW = CHUNK_ROWS * 4
OW = CHUNK_ROWS * 4


def _body(x_hbm, wb_hbm, vals_hbm, idx_hbm, wbuf, xbufs, vbufs, ibufs, sems):
    cid = lax.axis_index("c")
    sid = lax.axis_index("s")
    wid = sid * NC + cid

    pltpu.sync_copy(wb_hbm, wbuf)

    def bc(i):
        return wbuf[pl.ds(i * 16, 16)]

    def rne_bf16(v):
        u = plsc.bitcast(v, jnp.uint32)
        t = (u >> 16) & jnp.uint32(1)
        r = (u + jnp.uint32(0x7FFF) + t) & jnp.uint32(0xFFFF0000)
        return plsc.bitcast(r, jnp.float32)

    Wv = [[bc(j * 4 + k) for k in range(4)] for j in range(5)]
    Wbv = [[rne_bf16(Wv[m][k]) for k in range(4)] for m in range(5)]
    bv = [bc(20 + m) for m in range(5)]
    jconst = [jnp.full((16,), j, jnp.int32) for j in range(5)]
    zero = jnp.zeros((16,), jnp.float32)
    izero = jnp.zeros((16,), jnp.int32)

    def do_group(g, xbuf, vbuf, ibuf):
        tl = (g // 8) * 512
        u0 = (g % 8) * 16
        base = tl + u0

        s = []
        for j in range(5):
            xs = [
                rne_bf16(xbuf[pl.ds(j * XW + base + k * 128, 16)] + Wv[j][k])
                for k in range(4)
            ]
            sj = None
            for m in range(5):
                acc = xs[0] * Wbv[m][0]
                for k in range(1, 4):
                    acc = xs[k] * Wbv[m][k] + acc
                r = jnp.maximum(acc + bv[m], zero)
                sj = r if sj is None else sj + r
            s.append(sj)

        v = list(s)
        ix = list(jconst)

        def cex(p, q):
            c = v[q] > v[p]
            vp = jnp.where(c, v[q], v[p])
            vq = jnp.where(c, v[p], v[q])
            ip = jnp.where(c, ix[q], ix[p])
            iq = jnp.where(c, ix[p], ix[q])
            v[p], v[q] = vp, vq
            ix[p], ix[q] = ip, iq

        for p in (3, 2, 1, 0):
            cex(p, p + 1)
        for p in (3, 2, 1):
            cex(p, p + 1)
        for p in (3, 2):
            cex(p, p + 1)

        for t in range(3):
            vbuf[pl.ds(base + t * 128, 16)] = v[t]
            ibuf[pl.ds(base + t * 128, 16)] = ix[t]
        vbuf[pl.ds(base + 3 * 128, 16)] = zero
        ibuf[pl.ds(base + 3 * 128, 16)] = izero

    def start_in(c, buf):
        tile0 = (wid * ROWS_PER_W + c * CHUNK_ROWS) // 128
        cps = [
            pltpu.async_copy(
                x_hbm.at[pl.ds(j * (B * 4) + tile0 * 512, XW)],
                xbufs[buf].at[pl.ds(j * XW, XW)],
                sems[buf],
            )
            for j in range(5)
        ]
        return cps

    def start_out(c, buf):
        off = (wid * ROWS_PER_W + c * CHUNK_ROWS) * 4
        cv = pltpu.async_copy(vbufs[buf], vals_hbm.at[pl.ds(off, OW)],
                              sems[2 + buf])
        ci = pltpu.async_copy(ibufs[buf], idx_hbm.at[pl.ds(off, OW)],
                              sems[2 + buf])
        return cv, ci

    def compute(buf):
        def grp(g, carry):
            do_group(g, xbufs[buf], vbufs[buf], ibufs[buf])
            return carry

        lax.fori_loop(0, GROUPS, grp, 0)

    in_cps = {0: start_in(0, 0)}
    out_cps = {}
    for c in range(NCHUNK):
        buf = c % 2
        for cp in in_cps.pop(c):
            cp.wait()
        if c + 1 < NCHUNK:
            in_cps[c + 1] = start_in(c + 1, 1 - buf)
        if c - 2 in out_cps:
            for cp in out_cps.pop(c - 2):
                cp.wait()
        compute(buf)
        out_cps[c] = start_out(c, buf)
    for cps in out_cps.values():
        for cp in cps:
            cp.wait()


@jax.jit
def _run(xF, wb):
    mesh = plsc.VectorSubcoreMesh(
        core_axis_name="c", subcore_axis_name="s",
        num_cores=NC, num_subcores=NS,
    )
    f = pl.kernel(
        _body,
        out_type=[
            jax.ShapeDtypeStruct((B * 4,), jnp.float32),
            jax.ShapeDtypeStruct((B * 4,), jnp.int32),
        ],
        mesh=mesh,
        compiler_params=pltpu.CompilerParams(needs_layout_passes=False),
        scratch_types=[
            pltpu.VMEM((400,), jnp.float32),
            [pltpu.VMEM((5 * XW,), jnp.float32) for _ in range(2)],
            [pltpu.VMEM((OW,), jnp.float32) for _ in range(2)],
            [pltpu.VMEM((OW,), jnp.int32) for _ in range(2)],
            [pltpu.SemaphoreType.DMA for _ in range(4)],
        ],
    )
    return f(xF, wb)


def kernel(x, W, b):
    xF = jnp.transpose(x.reshape(B // 128, 128, 5, 4), (2, 0, 3, 1))
    xF = xF.reshape(B * 20)
    wb = jnp.repeat(jnp.concatenate([W.reshape(-1), b]), 16)
    vals4, idx4 = _run(xF, wb)
    vals = jnp.transpose(vals4.reshape(B // 128, 4, 128), (0, 2, 1))
    idxs = jnp.transpose(idx4.reshape(B // 128, 4, 128), (0, 2, 1))
    return (
        vals.reshape(B, 4)[:, :3],
        idxs.reshape(B, 4)[:, :3],
    )


# speedup vs baseline: 32.1162x; 1.1790x over previous
import jax
import jax.numpy as jnp
from jax import lax
from jax.experimental import pallas as pl
from jax.experimental.pallas import tpu as pltpu
from jax.experimental.pallas import tpu_sc as plsc

B = 1048576
NC = 2
NS = 16
NW = NC * NS
ROWS_PER_W = B // NW
CHUNK_ROWS = 2048
NCHUNK = ROWS_PER_W // CHUNK_ROWS
GROUPS = CHUNK_ROWS // 16
TILES = CHUNK_ROWS // 128
XW = CHUNK_ROWS * 4
OW = CHUNK_ROWS * 4


def _body(x_hbm, wb_hbm, vals_hbm, idx_hbm, wbuf, xbufs, vbufs, ibufs, sems):
    cid = lax.axis_index("c")
    sid = lax.axis_index("s")
    wid = sid * NC + cid

    pltpu.sync_copy(wb_hbm, wbuf)

    def bc(i):
        return wbuf[pl.ds(i * 16, 16)]

    def rne_bf16(v):
        u = plsc.bitcast(v, jnp.uint32)
        t = (u >> 16) & jnp.uint32(1)
        r = (u + jnp.uint32(0x7FFF) + t) & jnp.uint32(0xFFFF0000)
        return plsc.bitcast(r, jnp.float32)

    Wv = [[bc(j * 4 + k) for k in range(4)] for j in range(5)]
    Wbv = [[rne_bf16(Wv[m][k]) for k in range(4)] for m in range(5)]
    bv = [bc(20 + m) for m in range(5)]
    jconst = [jnp.full((16,), j, jnp.int32) for j in range(5)]
    zero = jnp.zeros((16,), jnp.float32)
    izero = jnp.zeros((16,), jnp.int32)

    def do_group(g, xbuf, vbuf, ibuf):
        tl = (g // 8) * 512
        u0 = (g % 8) * 16
        base = tl + u0

        s = []
        for j in range(5):
            xs = [
                rne_bf16(xbuf[pl.ds(j * XW + base + k * 128, 16)] + Wv[j][k])
                for k in range(4)
            ]
            sj = None
            for m in range(5):
                acc = xs[0] * Wbv[m][0]
                for k in range(1, 4):
                    acc = xs[k] * Wbv[m][k] + acc
                r = jnp.maximum(acc + bv[m], zero)
                sj = r if sj is None else sj + r
            s.append(sj)

        v = list(s)
        ix = list(jconst)

        def cex(p, q):
            c = v[q] > v[p]
            vp = jnp.where(c, v[q], v[p])
            vq = jnp.where(c, v[p], v[q])
            ip = jnp.where(c, ix[q], ix[p])
            iq = jnp.where(c, ix[p], ix[q])
            v[p], v[q] = vp, vq
            ix[p], ix[q] = ip, iq

        for p in (3, 2, 1, 0):
            cex(p, p + 1)
        for p in (3, 2, 1):
            cex(p, p + 1)
        for p in (3, 2):
            cex(p, p + 1)

        for t in range(3):
            vbuf[pl.ds(base + t * 128, 16)] = v[t]
            ibuf[pl.ds(base + t * 128, 16)] = ix[t]
        vbuf[pl.ds(base + 3 * 128, 16)] = zero
        ibuf[pl.ds(base + 3 * 128, 16)] = izero

    def start_in(c, buf):
        tile0 = (wid * ROWS_PER_W + c * CHUNK_ROWS) // 128
        for j in range(5):
            pltpu.async_copy(
                x_hbm.at[pl.ds(j * (B * 4) + tile0 * 512, XW)],
                xbufs[buf].at[pl.ds(j * XW, XW)],
                sems[buf],
            )

    def wait_in(buf):
        for j in range(5):
            pltpu.make_async_copy(
                x_hbm.at[pl.ds(j * (B * 4), XW)],
                xbufs[buf].at[pl.ds(j * XW, XW)],
                sems[buf],
            ).wait()

    def start_out(c, buf):
        off = (wid * ROWS_PER_W + c * CHUNK_ROWS) * 4
        pltpu.async_copy(vbufs[buf], vals_hbm.at[pl.ds(off, OW)],
                         sems[2 + buf])
        pltpu.async_copy(ibufs[buf], idx_hbm.at[pl.ds(off, OW)],
                         sems[2 + buf])

    def wait_out(buf):
        pltpu.make_async_copy(vbufs[buf], vals_hbm.at[pl.ds(0, OW)],
                              sems[2 + buf]).wait()
        pltpu.make_async_copy(ibufs[buf], idx_hbm.at[pl.ds(0, OW)],
                              sems[2 + buf]).wait()

    def compute(buf):
        def grp(g, carry):
            do_group(g, xbufs[buf], vbufs[buf], ibufs[buf])
            return carry

        lax.fori_loop(0, GROUPS, grp, 0)

    start_in(0, 0)

    def pipe(i, carry):
        c0 = i * 2
        wait_in(0)
        start_in(c0 + 1, 1)

        @pl.when(i > 0)
        def _():
            wait_out(0)

        compute(0)
        start_out(c0, 0)

        wait_in(1)

        @pl.when(c0 + 2 < NCHUNK)
        def _():
            start_in(c0 + 2, 0)

        @pl.when(i > 0)
        def _():
            wait_out(1)

        compute(1)
        start_out(c0 + 1, 1)
        return carry

    lax.fori_loop(0, NCHUNK // 2, pipe, 0)
    wait_out(0)
    wait_out(1)


@jax.jit
def _run(xF, wb):
    mesh = plsc.VectorSubcoreMesh(
        core_axis_name="c", subcore_axis_name="s",
        num_cores=NC, num_subcores=NS,
    )
    f = pl.kernel(
        _body,
        out_type=[
            jax.ShapeDtypeStruct((B * 4,), jnp.float32),
            jax.ShapeDtypeStruct((B * 4,), jnp.int32),
        ],
        mesh=mesh,
        compiler_params=pltpu.CompilerParams(needs_layout_passes=False),
        scratch_types=[
            pltpu.VMEM((400,), jnp.float32),
            [pltpu.VMEM((5 * XW,), jnp.float32) for _ in range(2)],
            [pltpu.VMEM((OW,), jnp.float32) for _ in range(2)],
            [pltpu.VMEM((OW,), jnp.int32) for _ in range(2)],
            [pltpu.SemaphoreType.DMA for _ in range(4)],
        ],
    )
    return f(xF, wb)


def kernel(x, W, b):
    xF = jnp.transpose(x.reshape(B // 128, 128, 5, 4), (2, 0, 3, 1))
    xF = xF.reshape(B * 20)
    wb = jnp.repeat(jnp.concatenate([W.reshape(-1), b]), 16)
    vals4, idx4 = _run(xF, wb)
    vals = jnp.transpose(vals4.reshape(B // 128, 4, 128), (0, 2, 1))
    idxs = jnp.transpose(idx4.reshape(B // 128, 4, 128), (0, 2, 1))
    return (
        vals.reshape(B, 4)[:, :3],
        idxs.reshape(B, 4)[:, :3],
    )


# speedup vs baseline: 34.6341x; 1.0784x over previous
import jax
import jax.numpy as jnp
from jax import lax
from jax.experimental import pallas as pl
from jax.experimental.pallas import tpu as pltpu
from jax.experimental.pallas import tpu_sc as plsc

B = 1048576
NC = 2
NS = 16
NW = NC * NS
ROWS_PER_W = B // NW
CHUNK_ROWS = 2048
NCHUNK = ROWS_PER_W // CHUNK_ROWS
GROUPS = CHUNK_ROWS // 16
TILES = CHUNK_ROWS // 128
XW = CHUNK_ROWS * 4
OW = CHUNK_ROWS * 4


def _body(x_hbm, wb_hbm, vals_hbm, idx_hbm, wbuf, xbufs, vbufs, ibufs, sems):
    cid = lax.axis_index("c")
    sid = lax.axis_index("s")
    wid = sid * NC + cid

    pltpu.sync_copy(wb_hbm, wbuf)

    def bc(i):
        return wbuf[pl.ds(i * 16, 16)]

    def rne_bf16(v):
        u = plsc.bitcast(v, jnp.uint32)
        t = (u >> 16) & jnp.uint32(1)
        r = (u + jnp.uint32(0x7FFF) + t) & jnp.uint32(0xFFFF0000)
        return plsc.bitcast(r, jnp.float32)

    Wv = [[bc(j * 4 + k) for k in range(4)] for j in range(5)]
    Wbv = [[rne_bf16(Wv[m][k]) for k in range(4)] for m in range(5)]
    bv = [bc(20 + m) for m in range(5)]
    jconst = [jnp.full((16,), j, jnp.int32) for j in range(5)]
    zero = jnp.zeros((16,), jnp.float32)
    izero = jnp.zeros((16,), jnp.int32)

    def do_group(base, xbuf, vbuf, ibuf):
        s = []
        for j in range(5):
            xs = [
                rne_bf16(xbuf[pl.ds(j * XW + base + k * 128, 16)] + Wv[j][k])
                for k in range(4)
            ]
            sj = None
            for m in range(5):
                acc = xs[0] * Wbv[m][0]
                for k in range(1, 4):
                    acc = xs[k] * Wbv[m][k] + acc
                r = jnp.maximum(acc + bv[m], zero)
                sj = r if sj is None else sj + r
            s.append(sj)

        v = list(s)
        ix = list(jconst)

        def cex(p, q, need_loser=True):
            c = v[q] > v[p]
            vp = jnp.where(c, v[q], v[p])
            ip = jnp.where(c, ix[q], ix[p])
            if need_loser:
                vq = jnp.where(c, v[p], v[q])
                iq = jnp.where(c, ix[p], ix[q])
                v[q], ix[q] = vq, iq
            v[p], ix[p] = vp, ip

        for p in (3, 2, 1, 0):
            cex(p, p + 1)
        for p in (3, 2, 1):
            cex(p, p + 1)
        cex(3, 4, need_loser=False)
        cex(2, 3, need_loser=False)

        for t in range(3):
            vbuf[pl.ds(base + t * 128, 16)] = v[t]
            ibuf[pl.ds(base + t * 128, 16)] = ix[t]
        vbuf[pl.ds(base + 3 * 128, 16)] = zero
        ibuf[pl.ds(base + 3 * 128, 16)] = izero

    def start_in(c, buf):
        tile0 = (wid * ROWS_PER_W + c * CHUNK_ROWS) // 128
        for j in range(5):
            pltpu.async_copy(
                x_hbm.at[pl.ds(j * (B * 4) + tile0 * 512, XW)],
                xbufs[buf].at[pl.ds(j * XW, XW)],
                sems[buf],
            )

    def wait_in(buf):
        for j in range(5):
            pltpu.make_async_copy(
                x_hbm.at[pl.ds(j * (B * 4), XW)],
                xbufs[buf].at[pl.ds(j * XW, XW)],
                sems[buf],
            ).wait()

    def start_out(c, buf):
        off = (wid * ROWS_PER_W + c * CHUNK_ROWS) * 4
        pltpu.async_copy(vbufs[buf], vals_hbm.at[pl.ds(off, OW)],
                         sems[2 + buf])
        pltpu.async_copy(ibufs[buf], idx_hbm.at[pl.ds(off, OW)],
                         sems[2 + buf])

    def wait_out(buf):
        pltpu.make_async_copy(vbufs[buf], vals_hbm.at[pl.ds(0, OW)],
                              sems[2 + buf]).wait()
        pltpu.make_async_copy(ibufs[buf], idx_hbm.at[pl.ds(0, OW)],
                              sems[2 + buf]).wait()

    def compute(buf):
        def grp(t, carry):
            tb = t * 512
            for gg in range(8):
                do_group(tb + gg * 16, xbufs[buf], vbufs[buf], ibufs[buf])
            return carry

        lax.fori_loop(0, TILES, grp, 0)

    start_in(0, 0)

    def pipe(i, carry):
        c0 = i * 2
        wait_in(0)
        start_in(c0 + 1, 1)

        @pl.when(i > 0)
        def _():
            wait_out(0)

        compute(0)
        start_out(c0, 0)

        wait_in(1)

        @pl.when(c0 + 2 < NCHUNK)
        def _():
            start_in(c0 + 2, 0)

        @pl.when(i > 0)
        def _():
            wait_out(1)

        compute(1)
        start_out(c0 + 1, 1)
        return carry

    lax.fori_loop(0, NCHUNK // 2, pipe, 0)
    wait_out(0)
    wait_out(1)


@jax.jit
def _run(xF, wb):
    mesh = plsc.VectorSubcoreMesh(
        core_axis_name="c", subcore_axis_name="s",
        num_cores=NC, num_subcores=NS,
    )
    f = pl.kernel(
        _body,
        out_type=[
            jax.ShapeDtypeStruct((B * 4,), jnp.float32),
            jax.ShapeDtypeStruct((B * 4,), jnp.int32),
        ],
        mesh=mesh,
        compiler_params=pltpu.CompilerParams(needs_layout_passes=False),
        scratch_types=[
            pltpu.VMEM((400,), jnp.float32),
            [pltpu.VMEM((5 * XW,), jnp.float32) for _ in range(2)],
            [pltpu.VMEM((OW,), jnp.float32) for _ in range(2)],
            [pltpu.VMEM((OW,), jnp.int32) for _ in range(2)],
            [pltpu.SemaphoreType.DMA for _ in range(4)],
        ],
    )
    return f(xF, wb)


def kernel(x, W, b):
    xF = jnp.transpose(x.reshape(B // 128, 128, 5, 4), (2, 0, 3, 1))
    xF = xF.reshape(B * 20)
    wb = jnp.repeat(jnp.concatenate([W.reshape(-1), b]), 16)
    vals4, idx4 = _run(xF, wb)
    vals = jnp.transpose(vals4.reshape(B // 128, 4, 128), (0, 2, 1))
    idxs = jnp.transpose(idx4.reshape(B // 128, 4, 128), (0, 2, 1))
    return (
        vals.reshape(B, 4)[:, :3],
        idxs.reshape(B, 4)[:, :3],
    )


# speedup vs baseline: 40.2430x; 1.1619x over previous
import jax
import jax.numpy as jnp
from jax import lax
from jax.experimental import pallas as pl
from jax.experimental.pallas import tpu as pltpu
from jax.experimental.pallas import tpu_sc as plsc

B = 1048576
NC = 2
NS = 16
NW = NC * NS
ROWS_PER_W = B // NW
CHUNK_ROWS = 2048
NCHUNK = ROWS_PER_W // CHUNK_ROWS
GROUPS = CHUNK_ROWS // 16
TILES = CHUNK_ROWS // 128
XW = CHUNK_ROWS * 4
OW = CHUNK_ROWS * 4


def _body(x_hbm, wb_hbm, vals_hbm, idx_hbm, wbuf, xbufs, vbufs, ibufs, sems):
    cid = lax.axis_index("c")
    sid = lax.axis_index("s")
    wid = sid * NC + cid

    pltpu.sync_copy(wb_hbm, wbuf)

    def bc(i):
        return wbuf[pl.ds(i * 16, 16)]

    def rne_bf16(v):
        u = plsc.bitcast(v, jnp.uint32)
        t = (u >> 16) & jnp.uint32(1)
        r = (u + jnp.uint32(0x7FFF) + t) & jnp.uint32(0xFFFF0000)
        return plsc.bitcast(r, jnp.float32)

    jconst = [jnp.full((16,), j, jnp.int32) for j in range(5)]
    zero = jnp.zeros((16,), jnp.float32)
    izero = jnp.zeros((16,), jnp.int32)

    def do_group(base, xbuf, vbuf, ibuf):
        xs = [
            [
                rne_bf16(
                    xbuf[pl.ds(j * XW + base + k * 128, 16)] + bc(j * 4 + k)
                )
                for k in range(4)
            ]
            for j in range(5)
        ]
        s = [None] * 5
        for m in range(5):
            wbm = [bc(20 + m * 4 + k) for k in range(4)]
            bvm = bc(40 + m)
            for j in range(5):
                acc = xs[j][0] * wbm[0]
                for k in range(1, 4):
                    acc = xs[j][k] * wbm[k] + acc
                r = jnp.maximum(acc + bvm, zero)
                s[j] = r if s[j] is None else s[j] + r

        v = list(s)
        ix = list(jconst)

        def cex(p, q, need_loser=True):
            c = v[q] > v[p]
            vp = jnp.where(c, v[q], v[p])
            ip = jnp.where(c, ix[q], ix[p])
            if need_loser:
                vq = jnp.where(c, v[p], v[q])
                iq = jnp.where(c, ix[p], ix[q])
                v[q], ix[q] = vq, iq
            v[p], ix[p] = vp, ip

        for p in (3, 2, 1, 0):
            cex(p, p + 1)
        for p in (3, 2, 1):
            cex(p, p + 1)
        cex(3, 4, need_loser=False)
        cex(2, 3, need_loser=False)

        for t in range(3):
            vbuf[pl.ds(base + t * 128, 16)] = v[t]
            ibuf[pl.ds(base + t * 128, 16)] = ix[t]
        vbuf[pl.ds(base + 3 * 128, 16)] = zero
        ibuf[pl.ds(base + 3 * 128, 16)] = izero

    def start_in(c, buf):
        tile0 = (wid * ROWS_PER_W + c * CHUNK_ROWS) // 128
        for j in range(5):
            pltpu.async_copy(
                x_hbm.at[pl.ds(j * (B * 4) + tile0 * 512, XW)],
                xbufs[buf].at[pl.ds(j * XW, XW)],
                sems[buf],
            )

    def wait_in(buf):
        for j in range(5):
            pltpu.make_async_copy(
                x_hbm.at[pl.ds(j * (B * 4), XW)],
                xbufs[buf].at[pl.ds(j * XW, XW)],
                sems[buf],
            ).wait()

    def start_out(c, buf):
        off = (wid * ROWS_PER_W + c * CHUNK_ROWS) * 4
        pltpu.async_copy(vbufs[buf], vals_hbm.at[pl.ds(off, OW)],
                         sems[2 + buf])
        pltpu.async_copy(ibufs[buf], idx_hbm.at[pl.ds(off, OW)],
                         sems[2 + buf])

    def wait_out(buf):
        pltpu.make_async_copy(vbufs[buf], vals_hbm.at[pl.ds(0, OW)],
                              sems[2 + buf]).wait()
        pltpu.make_async_copy(ibufs[buf], idx_hbm.at[pl.ds(0, OW)],
                              sems[2 + buf]).wait()

    def compute(buf):
        def grp(t, carry):
            tb = t * 512
            for gg in range(8):
                do_group(tb + gg * 16, xbufs[buf], vbufs[buf], ibufs[buf])
            return carry

        lax.fori_loop(0, TILES, grp, 0)

    start_in(0, 0)

    def pipe(i, carry):
        c0 = i * 2
        wait_in(0)
        start_in(c0 + 1, 1)

        @pl.when(i > 0)
        def _():
            wait_out(0)

        compute(0)
        start_out(c0, 0)

        wait_in(1)

        @pl.when(c0 + 2 < NCHUNK)
        def _():
            start_in(c0 + 2, 0)

        @pl.when(i > 0)
        def _():
            wait_out(1)

        compute(1)
        start_out(c0 + 1, 1)
        return carry

    lax.fori_loop(0, NCHUNK // 2, pipe, 0)
    wait_out(0)
    wait_out(1)


@jax.jit
def _run(xF, wb):
    mesh = plsc.VectorSubcoreMesh(
        core_axis_name="c", subcore_axis_name="s",
        num_cores=NC, num_subcores=NS,
    )
    f = pl.kernel(
        _body,
        out_type=[
            jax.ShapeDtypeStruct((B * 4,), jnp.float32),
            jax.ShapeDtypeStruct((B * 4,), jnp.int32),
        ],
        mesh=mesh,
        compiler_params=pltpu.CompilerParams(needs_layout_passes=False),
        scratch_types=[
            pltpu.VMEM((720,), jnp.float32),
            [pltpu.VMEM((5 * XW,), jnp.float32) for _ in range(2)],
            [pltpu.VMEM((OW,), jnp.float32) for _ in range(2)],
            [pltpu.VMEM((OW,), jnp.int32) for _ in range(2)],
            [pltpu.SemaphoreType.DMA for _ in range(4)],
        ],
    )
    return f(xF, wb)


def kernel(x, W, b):
    xF = jnp.transpose(x.reshape(B // 128, 128, 5, 4), (2, 0, 3, 1))
    xF = xF.reshape(B * 20)
    wu = lax.bitcast_convert_type(W.reshape(-1), jnp.uint32)
    wt = (wu >> 16) & jnp.uint32(1)
    wr = (wu + jnp.uint32(0x7FFF) + wt) & jnp.uint32(0xFFFF0000)
    Wbf = lax.bitcast_convert_type(wr, jnp.float32)
    wb = jnp.repeat(jnp.concatenate([W.reshape(-1), Wbf, b]), 16)
    vals4, idx4 = _run(xF, wb)
    vals = jnp.transpose(vals4.reshape(B // 128, 4, 128), (0, 2, 1))
    idxs = jnp.transpose(idx4.reshape(B // 128, 4, 128), (0, 2, 1))
    return (
        vals.reshape(B, 4)[:, :3],
        idxs.reshape(B, 4)[:, :3],
    )


# speedup vs baseline: 42.8526x; 1.0648x over previous
import jax
import jax.numpy as jnp
from jax import lax
from jax.experimental import pallas as pl
from jax.experimental.pallas import tpu as pltpu
from jax.experimental.pallas import tpu_sc as plsc

B = 1048576
NC = 2
NS = 16
NW = NC * NS
ROWS_PER_W = B // NW
CHUNK_ROWS = 2048
NCHUNK = ROWS_PER_W // CHUNK_ROWS
GROUPS = CHUNK_ROWS // 16
TILES = CHUNK_ROWS // 128
XW = CHUNK_ROWS * 4
OW = CHUNK_ROWS * 4


def _body(x_hbm, wb_hbm, vals_hbm, idx_hbm, wbuf, xbufs, vbufs, ibufs, sems):
    cid = lax.axis_index("c")
    sid = lax.axis_index("s")
    wid = sid * NC + cid

    pltpu.sync_copy(wb_hbm, wbuf)

    def bc(i):
        return wbuf[pl.ds(i * 16, 16)]

    vkC = jnp.full((16,), 65537.0, jnp.float32)

    def rne_bf16(v):
        sp = v * vkC
        return sp - (sp - v)

    jconst = [jnp.full((16,), j, jnp.int32) for j in range(5)]
    zero = jnp.zeros((16,), jnp.float32)
    izero = jnp.zeros((16,), jnp.int32)

    def do_group(base, xbuf, vbuf, ibuf):
        xs = [
            [
                rne_bf16(
                    xbuf[pl.ds(j * XW + base + k * 128, 16)] + bc(j * 4 + k)
                )
                for k in range(4)
            ]
            for j in range(5)
        ]
        s = [None] * 5
        for m in range(5):
            wbm = [bc(20 + m * 4 + k) for k in range(4)]
            bvm = bc(40 + m)
            for j in range(5):
                acc = xs[j][0] * wbm[0]
                for k in range(1, 4):
                    acc = xs[j][k] * wbm[k] + acc
                r = jnp.maximum(acc + bvm, zero)
                s[j] = r if s[j] is None else s[j] + r

        v = list(s)
        ix = list(jconst)

        def cex(p, q, need_loser=True):
            c = v[q] > v[p]
            vp = jnp.where(c, v[q], v[p])
            ip = jnp.where(c, ix[q], ix[p])
            if need_loser:
                vq = jnp.where(c, v[p], v[q])
                iq = jnp.where(c, ix[p], ix[q])
                v[q], ix[q] = vq, iq
            v[p], ix[p] = vp, ip

        for p in (3, 2, 1, 0):
            cex(p, p + 1)
        for p in (3, 2, 1):
            cex(p, p + 1)
        cex(3, 4, need_loser=False)
        cex(2, 3, need_loser=False)

        for t in range(3):
            vbuf[pl.ds(base + t * 128, 16)] = v[t]
            ibuf[pl.ds(base + t * 128, 16)] = ix[t]
        vbuf[pl.ds(base + 3 * 128, 16)] = zero
        ibuf[pl.ds(base + 3 * 128, 16)] = izero

    def start_in(c, buf):
        tile0 = (wid * ROWS_PER_W + c * CHUNK_ROWS) // 128
        for j in range(5):
            pltpu.async_copy(
                x_hbm.at[pl.ds(j * (B * 4) + tile0 * 512, XW)],
                xbufs[buf].at[pl.ds(j * XW, XW)],
                sems[buf],
            )

    def wait_in(buf):
        for j in range(5):
            pltpu.make_async_copy(
                x_hbm.at[pl.ds(j * (B * 4), XW)],
                xbufs[buf].at[pl.ds(j * XW, XW)],
                sems[buf],
            ).wait()

    def start_out(c, buf):
        off = (wid * ROWS_PER_W + c * CHUNK_ROWS) * 4
        pltpu.async_copy(vbufs[buf], vals_hbm.at[pl.ds(off, OW)],
                         sems[2 + buf])
        pltpu.async_copy(ibufs[buf], idx_hbm.at[pl.ds(off, OW)],
                         sems[2 + buf])

    def wait_out(buf):
        pltpu.make_async_copy(vbufs[buf], vals_hbm.at[pl.ds(0, OW)],
                              sems[2 + buf]).wait()
        pltpu.make_async_copy(ibufs[buf], idx_hbm.at[pl.ds(0, OW)],
                              sems[2 + buf]).wait()

    def compute(buf):
        def grp(t, carry):
            tb = t * 512
            for gg in range(8):
                do_group(tb + gg * 16, xbufs[buf], vbufs[buf], ibufs[buf])
            return carry

        lax.fori_loop(0, TILES, grp, 0)

    start_in(0, 0)

    def pipe(i, carry):
        c0 = i * 2
        wait_in(0)
        start_in(c0 + 1, 1)

        @pl.when(i > 0)
        def _():
            wait_out(0)

        compute(0)
        start_out(c0, 0)

        wait_in(1)

        @pl.when(c0 + 2 < NCHUNK)
        def _():
            start_in(c0 + 2, 0)

        @pl.when(i > 0)
        def _():
            wait_out(1)

        compute(1)
        start_out(c0 + 1, 1)
        return carry

    lax.fori_loop(0, NCHUNK // 2, pipe, 0)
    wait_out(0)
    wait_out(1)


@jax.jit
def _run(xF, wb):
    mesh = plsc.VectorSubcoreMesh(
        core_axis_name="c", subcore_axis_name="s",
        num_cores=NC, num_subcores=NS,
    )
    f = pl.kernel(
        _body,
        out_type=[
            jax.ShapeDtypeStruct((B * 4,), jnp.float32),
            jax.ShapeDtypeStruct((B * 4,), jnp.int32),
        ],
        mesh=mesh,
        compiler_params=pltpu.CompilerParams(needs_layout_passes=False),
        scratch_types=[
            pltpu.VMEM((720,), jnp.float32),
            [pltpu.VMEM((5 * XW,), jnp.float32) for _ in range(2)],
            [pltpu.VMEM((OW,), jnp.float32) for _ in range(2)],
            [pltpu.VMEM((OW,), jnp.int32) for _ in range(2)],
            [pltpu.SemaphoreType.DMA for _ in range(4)],
        ],
    )
    return f(xF, wb)


def kernel(x, W, b):
    xF = jnp.transpose(x.reshape(B // 128, 128, 5, 4), (2, 0, 3, 1))
    xF = xF.reshape(B * 20)
    wu = lax.bitcast_convert_type(W.reshape(-1), jnp.uint32)
    wt = (wu >> 16) & jnp.uint32(1)
    wr = (wu + jnp.uint32(0x7FFF) + wt) & jnp.uint32(0xFFFF0000)
    Wbf = lax.bitcast_convert_type(wr, jnp.float32)
    wb = jnp.repeat(jnp.concatenate([W.reshape(-1), Wbf, b]), 16)
    vals4, idx4 = _run(xF, wb)
    vals = jnp.transpose(vals4.reshape(B // 128, 4, 128), (0, 2, 1))
    idxs = jnp.transpose(idx4.reshape(B // 128, 4, 128), (0, 2, 1))
    return (
        vals.reshape(B, 4)[:, :3],
        idxs.reshape(B, 4)[:, :3],
    )


# speedup vs baseline: 42.8630x; 1.0002x over previous
import jax
import jax.numpy as jnp
from jax import lax
from jax.experimental import pallas as pl
from jax.experimental.pallas import tpu as pltpu
from jax.experimental.pallas import tpu_sc as plsc

B = 1048576
NC = 2
NS = 16
NW = NC * NS
ROWS_PER_W = B // NW
CHUNK_ROWS = 2048
NCHUNK = ROWS_PER_W // CHUNK_ROWS
GROUPS = CHUNK_ROWS // 16
TILES = CHUNK_ROWS // 128
XW = CHUNK_ROWS * 4
OW = CHUNK_ROWS * 4


def _body(x_hbm, wb_hbm, vals_hbm, idx_hbm, wbuf, xbufs, vbufs, ibufs, sems):
    cid = lax.axis_index("c")
    sid = lax.axis_index("s")
    wid = sid * NC + cid

    pltpu.sync_copy(wb_hbm, wbuf)

    def bc(i):
        return wbuf[pl.ds(i * 16, 16)]

    vkC = jnp.full((16,), 65537.0, jnp.float32)

    def rne_bf16(v):
        sp = v * vkC
        return sp - (sp - v)

    jconst = [jnp.full((16,), j, jnp.int32) for j in range(5)]
    zero = jnp.zeros((16,), jnp.float32)
    izero = jnp.zeros((16,), jnp.int32)

    def do_group(base, xbuf, vbuf, ibuf):
        xs = [
            [
                rne_bf16(
                    xbuf[pl.ds(j * XW + base + k * 128, 16)] + bc(j * 4 + k)
                )
                for k in range(4)
            ]
            for j in range(5)
        ]
        s = [None] * 5
        for m in range(5):
            wbm = [bc(20 + m * 4 + k) for k in range(4)]
            bvm = bc(40 + m)
            for j in range(5):
                acc = xs[j][0] * wbm[0]
                for k in range(1, 4):
                    acc = xs[j][k] * wbm[k] + acc
                r = jnp.maximum(acc + bvm, zero)
                s[j] = r if s[j] is None else s[j] + r

        v = list(s)
        ix = list(jconst)

        def cex(p, q, need_loser=True):
            c = v[q] > v[p]
            vp = jnp.where(c, v[q], v[p])
            ip = jnp.where(c, ix[q], ix[p])
            if need_loser:
                vq = jnp.where(c, v[p], v[q])
                iq = jnp.where(c, ix[p], ix[q])
                v[q], ix[q] = vq, iq
            v[p], ix[p] = vp, ip

        for p in (3, 2, 1, 0):
            cex(p, p + 1)
        for p in (3, 2, 1):
            cex(p, p + 1)
        cex(3, 4, need_loser=False)
        cex(2, 3, need_loser=False)

        for t in range(3):
            vbuf[pl.ds(base + t * 128, 16)] = v[t]
            ibuf[pl.ds(base + t * 128, 16)] = ix[t]
        vbuf[pl.ds(base + 3 * 128, 16)] = zero
        ibuf[pl.ds(base + 3 * 128, 16)] = izero

    def start_in(c, buf):
        tile0 = (wid * ROWS_PER_W + c * CHUNK_ROWS) // 128
        for j in range(5):
            pltpu.async_copy(
                x_hbm.at[pl.ds(j * (B * 4) + tile0 * 512, XW)],
                xbufs[buf].at[pl.ds(j * XW, XW)],
                sems[buf],
            )

    def wait_in(buf):
        for j in range(5):
            pltpu.make_async_copy(
                x_hbm.at[pl.ds(j * (B * 4), XW)],
                xbufs[buf].at[pl.ds(j * XW, XW)],
                sems[buf],
            ).wait()

    def start_out(c, buf):
        off = (wid * ROWS_PER_W + c * CHUNK_ROWS) * 4
        pltpu.async_copy(vbufs[buf], vals_hbm.at[pl.ds(off, OW)],
                         sems[2 + buf])
        pltpu.async_copy(ibufs[buf], idx_hbm.at[pl.ds(off, OW)],
                         sems[2 + buf])

    def wait_out(buf):
        pltpu.make_async_copy(vbufs[buf], vals_hbm.at[pl.ds(0, OW)],
                              sems[2 + buf]).wait()
        pltpu.make_async_copy(ibufs[buf], idx_hbm.at[pl.ds(0, OW)],
                              sems[2 + buf]).wait()

    def compute(buf):
        @plsc.parallel_loop(0, TILES, step=1)
        def grp(t):
            tb = t * 512
            for gg in range(8):
                do_group(tb + gg * 16, xbufs[buf], vbufs[buf], ibufs[buf])

    start_in(0, 0)

    def pipe(i, carry):
        c0 = i * 2
        wait_in(0)
        start_in(c0 + 1, 1)

        @pl.when(i > 0)
        def _():
            wait_out(0)

        compute(0)
        start_out(c0, 0)

        wait_in(1)

        @pl.when(c0 + 2 < NCHUNK)
        def _():
            start_in(c0 + 2, 0)

        @pl.when(i > 0)
        def _():
            wait_out(1)

        compute(1)
        start_out(c0 + 1, 1)
        return carry

    lax.fori_loop(0, NCHUNK // 2, pipe, 0)
    wait_out(0)
    wait_out(1)


@jax.jit
def _run(xF, wb):
    mesh = plsc.VectorSubcoreMesh(
        core_axis_name="c", subcore_axis_name="s",
        num_cores=NC, num_subcores=NS,
    )
    f = pl.kernel(
        _body,
        out_type=[
            jax.ShapeDtypeStruct((B * 4,), jnp.float32),
            jax.ShapeDtypeStruct((B * 4,), jnp.int32),
        ],
        mesh=mesh,
        compiler_params=pltpu.CompilerParams(needs_layout_passes=False),
        scratch_types=[
            pltpu.VMEM((720,), jnp.float32),
            [pltpu.VMEM((5 * XW,), jnp.float32) for _ in range(2)],
            [pltpu.VMEM((OW,), jnp.float32) for _ in range(2)],
            [pltpu.VMEM((OW,), jnp.int32) for _ in range(2)],
            [pltpu.SemaphoreType.DMA for _ in range(4)],
        ],
    )
    return f(xF, wb)


def kernel(x, W, b):
    xF = jnp.transpose(x.reshape(B // 128, 128, 5, 4), (2, 0, 3, 1))
    xF = xF.reshape(B * 20)
    wu = lax.bitcast_convert_type(W.reshape(-1), jnp.uint32)
    wt = (wu >> 16) & jnp.uint32(1)
    wr = (wu + jnp.uint32(0x7FFF) + wt) & jnp.uint32(0xFFFF0000)
    Wbf = lax.bitcast_convert_type(wr, jnp.float32)
    wb = jnp.repeat(jnp.concatenate([W.reshape(-1), Wbf, b]), 16)
    vals4, idx4 = _run(xF, wb)
    vals = jnp.transpose(vals4.reshape(B // 128, 4, 128), (0, 2, 1))
    idxs = jnp.transpose(idx4.reshape(B // 128, 4, 128), (0, 2, 1))
    return (
        vals.reshape(B, 4)[:, :3],
        idxs.reshape(B, 4)[:, :3],
    )


# speedup vs baseline: 42.9615x; 1.0023x over previous
import jax
import jax.numpy as jnp
from jax import lax
from jax.experimental import pallas as pl
from jax.experimental.pallas import tpu as pltpu
from jax.experimental.pallas import tpu_sc as plsc

B = 1048576
NC = 2
NS = 16
NW = NC * NS
ROWS_PER_W = B // NW
CHUNK_ROWS = 2048
NCHUNK = ROWS_PER_W // CHUNK_ROWS
TILES = CHUNK_ROWS // 128
XW = CHUNK_ROWS * 4
OW = CHUNK_ROWS * 4


def _body(x_hbm, wb_hbm, vals_hbm, idx_hbm, wbuf, xbufs, vbufs, ibufs, sems):
    cid = lax.axis_index("c")
    sid = lax.axis_index("s")
    wid = sid * NC + cid

    pltpu.sync_copy(wb_hbm, wbuf)

    def bc(i):
        return wbuf[pl.ds(i * 16, 16)]

    vkC = jnp.full((16,), 65537.0, jnp.float32)

    def rne_bf16(v):
        sp = v * vkC
        return sp - (sp - v)

    jconst = [jnp.full((16,), j, jnp.int32) for j in range(5)]
    zero = jnp.zeros((16,), jnp.float32)
    izero = jnp.zeros((16,), jnp.int32)

    def do_group(base, xbuf, vbuf, ibuf):
        xs = [
            [
                rne_bf16(
                    xbuf[pl.ds(j * XW + base + k * 128, 16)] + bc(j * 4 + k)
                )
                for k in range(4)
            ]
            for j in range(5)
        ]
        s = [None] * 5
        for m in range(5):
            wbm = [bc(20 + m * 4 + k) for k in range(4)]
            bvm = bc(40 + m)
            for j in range(5):
                acc = xs[j][0] * wbm[0]
                for k in range(1, 4):
                    acc = xs[j][k] * wbm[k] + acc
                r = jnp.maximum(acc + bvm, zero)
                s[j] = r if s[j] is None else s[j] + r

        v = list(s)
        ix = list(jconst)

        def cex(p, q, need_loser=True):
            c = v[q] > v[p]
            vp = jnp.where(c, v[q], v[p])
            ip = jnp.where(c, ix[q], ix[p])
            if need_loser:
                vq = jnp.where(c, v[p], v[q])
                iq = jnp.where(c, ix[p], ix[q])
                v[q], ix[q] = vq, iq
            v[p], ix[p] = vp, ip

        for p in (3, 2, 1, 0):
            cex(p, p + 1)
        for p in (3, 2, 1):
            cex(p, p + 1)
        cex(3, 4, need_loser=False)
        cex(2, 3, need_loser=False)

        for t in range(3):
            vbuf[pl.ds(base + t * 128, 16)] = v[t]
            ibuf[pl.ds(base + t * 128, 16)] = ix[t]
        vbuf[pl.ds(base + 3 * 128, 16)] = zero
        ibuf[pl.ds(base + 3 * 128, 16)] = izero

    def start_in(c, buf):
        tile0 = (wid * ROWS_PER_W + c * CHUNK_ROWS) // 128
        for j in range(5):
            pltpu.async_copy(
                x_hbm.at[pl.ds(j * (B * 4) + tile0 * 512, XW)],
                xbufs[buf].at[pl.ds(j * XW, XW)],
                sems[buf],
            )

    def wait_in(buf):
        for j in range(5):
            pltpu.make_async_copy(
                x_hbm.at[pl.ds(j * (B * 4), XW)],
                xbufs[buf].at[pl.ds(j * XW, XW)],
                sems[buf],
            ).wait()

    def start_out(c, buf):
        off = (wid * ROWS_PER_W + c * CHUNK_ROWS) * 4
        pltpu.async_copy(vbufs[buf], vals_hbm.at[pl.ds(off, OW)],
                         sems[2 + buf])
        pltpu.async_copy(ibufs[buf], idx_hbm.at[pl.ds(off, OW)],
                         sems[2 + buf])

    def wait_out(buf):
        pltpu.make_async_copy(vbufs[buf], vals_hbm.at[pl.ds(0, OW)],
                              sems[2 + buf]).wait()
        pltpu.make_async_copy(ibufs[buf], idx_hbm.at[pl.ds(0, OW)],
                              sems[2 + buf]).wait()

    def compute(buf):
        @plsc.parallel_loop(0, TILES, step=1)
        def grp(t):
            tb = t * 512
            for gg in range(8):
                do_group(tb + gg * 16, xbufs[buf], vbufs[buf], ibufs[buf])

    start_in(0, 0)

    def pipe(i, carry):
        c0 = i * 2
        wait_in(0)
        start_in(c0 + 1, 1)

        @pl.when(i > 0)
        def _():
            wait_out(0)

        compute(0)
        start_out(c0, 0)

        wait_in(1)

        @pl.when(c0 + 2 < NCHUNK)
        def _():
            start_in(c0 + 2, 0)

        @pl.when(i > 0)
        def _():
            wait_out(1)

        compute(1)
        start_out(c0 + 1, 1)
        return carry

    lax.fori_loop(0, NCHUNK // 2, pipe, 0)
    wait_out(0)
    wait_out(1)


@jax.jit
def _run(xF, wb):
    mesh = plsc.VectorSubcoreMesh(
        core_axis_name="c", subcore_axis_name="s",
        num_cores=NC, num_subcores=NS,
    )
    f = pl.kernel(
        _body,
        out_type=[
            jax.ShapeDtypeStruct((B * 4,), jnp.float32),
            jax.ShapeDtypeStruct((B * 4,), jnp.int32),
        ],
        mesh=mesh,
        compiler_params=pltpu.CompilerParams(needs_layout_passes=False),
        scratch_types=[
            pltpu.VMEM((720,), jnp.float32),
            [pltpu.VMEM((5 * XW,), jnp.float32) for _ in range(2)],
            [pltpu.VMEM((OW,), jnp.float32) for _ in range(2)],
            [pltpu.VMEM((OW,), jnp.int32) for _ in range(2)],
            [pltpu.SemaphoreType.DMA for _ in range(4)],
        ],
    )
    return f(xF, wb)


def kernel(x, W, b):
    xF = jnp.transpose(x.reshape(B // 128, 128, 5, 4), (2, 0, 3, 1))
    xF = xF.reshape(B * 20)
    wu = lax.bitcast_convert_type(W.reshape(-1), jnp.uint32)
    wt = (wu >> 16) & jnp.uint32(1)
    wr = (wu + jnp.uint32(0x7FFF) + wt) & jnp.uint32(0xFFFF0000)
    Wbf = lax.bitcast_convert_type(wr, jnp.float32)
    wb = jnp.repeat(jnp.concatenate([W.reshape(-1), Wbf, b]), 16)
    vals4, idx4 = _run(xF, wb)
    vals = jnp.transpose(vals4.reshape(B // 128, 4, 128), (0, 2, 1))
    idxs = jnp.transpose(idx4.reshape(B // 128, 4, 128), (0, 2, 1))
    return (
        vals.reshape(B, 4)[:, :3],
        idxs.reshape(B, 4)[:, :3],
    )
